# baseline (speedup 1.0000x reference)
import functools
import math

import jax
import jax.numpy as jnp
from jax import lax
from jax.experimental import pallas as pl
from jax.experimental.pallas import tpu as pltpu
from jax.experimental.pallas import tpu_sc as plsc

_C = 100000
_N = 1024
_SMOOTHING = 0.1
_CONF = 1.0 - _SMOOTHING
_FILL = _SMOOTHING / (_C - 1)
_CONST = (_C - 1) * _FILL * math.log(_FILL) + _CONF * math.log(_CONF)
_WF = -_FILL / _N
_WC = -_CONF / _N

_NTC = 768
_NSC = _N - _NTC

_BR = 8
_NSL = (_C + 127) // 128
_BC = _NSL * 128
_NRG = _NTC // _BR
_NACC = 8

_NW = 32
_RPW = _NSC // _NW
_CTAIL = (_C // 128) * 128
_CW = 3072
_NFULL = _CTAIL // _CW
_CREM = _CTAIL - _NFULL * _CW
_NCH = _NFULL + 1
_GSAFE = _CTAIL - 128


def _accum(buf, width, acc0, acc1):
    def body(k, carry):
        a0, a1 = carry
        base = k * 16
        for rr in range(_RPW):
            v = buf[rr, pl.ds(base, 16)]
            if rr % 2 == 0:
                a0 = a0 + v
            else:
                a1 = a1 + v
        return (a0, a1)
    return lax.fori_loop(0, width // 16, body, (acc0, acc1))


@functools.partial(
    pl.kernel,
    out_type=jax.ShapeDtypeStruct((_NW, 16), jnp.float32),
    scratch_types=[
        pltpu.VMEM((_RPW, _CW), jnp.float32),
        pltpu.VMEM((_RPW, _CW), jnp.float32),
        pltpu.VMEM((_RPW, 128), jnp.int32),
        pltpu.VMEM((_RPW, _RPW, 128), jnp.float32),
        pltpu.VMEM((16,), jnp.float32),
        pltpu.SemaphoreType.DMA,
        pltpu.SemaphoreType.DMA,
        pltpu.SemaphoreType.DMA,
    ],
    mesh=plsc.VectorSubcoreMesh(core_axis_name="c", subcore_axis_name="s"),
)
def _sc_part(x_hbm, t128_hbm, out_hbm, buf0, buf1, idx_v, vals_v, accv,
             sem0, sem1, semg):
    wid = lax.axis_index("s") * 2 + lax.axis_index("c")
    r0 = wid * _RPW
    bufs = (buf0, buf1)
    sems = (sem0, sem1)

    pltpu.sync_copy(t128_hbm.at[pl.ds(r0, _RPW)], idx_v)

    gcopies = []
    for r in range(_RPW):
        t_s = idx_v[r, pl.ds(0, 16)][0]
        base = pl.multiple_of(
            jnp.minimum((t_s // 128) * 128, _GSAFE), 128)
        gcopies.append(pltpu.async_copy(
            x_hbm.at[pl.ds(r0, _RPW), pl.ds(base, 128)],
            vals_v.at[r],
            semg))

    def chunk_src(c):
        w = _CW if c < _NFULL else _CREM
        return x_hbm.at[pl.ds(r0, _RPW), pl.ds(c * _CW, w)]

    def chunk_dst(c):
        if c < _NFULL:
            return bufs[c % 2]
        return bufs[c % 2].at[:, pl.ds(0, _CREM)]

    copies = {0: pltpu.async_copy(chunk_src(0), chunk_dst(0), sems[0])}
    acc0 = jnp.zeros((16,), jnp.float32)
    acc1 = jnp.zeros((16,), jnp.float32)
    for c in range(_NCH):
        if c + 1 < _NCH:
            copies[c + 1] = pltpu.async_copy(
                chunk_src(c + 1), chunk_dst(c + 1), sems[(c + 1) % 2])
        copies[c].wait()
        w = _CW if c < _NFULL else _CREM
        acc0, acc1 = _accum(bufs[c % 2], w, acc0, acc1)

    for cp in gcopies:
        cp.wait()
    lane16 = lax.iota(jnp.int32, 16)
    gacc = jnp.zeros((16,), jnp.float32)
    for r in range(_RPW):
        t_s = idx_v[r, pl.ds(0, 16)][0]
        woff = ((t_s % 128) // 16) * 16
        vec = vals_v[r, r, pl.ds(woff, 16)]
        vec = jnp.where(t_s < _CTAIL, vec, 0.0)
        gacc = gacc + jnp.where(lane16 == t_s % 16, vec, 0.0)

    accv[...] = (jnp.float32(_WF) * (acc0 + acc1)
                 + jnp.float32(_WC - _WF) * gacc)
    pltpu.sync_copy(accv, out_hbm.at[wid])


def _tc_body(t_ref, x_ref, tt_ref, xt_ref, o_ref, acc_ref):
    i = pl.program_id(0)

    @pl.when(i == 0)
    def _init():
        acc_ref[...] = jnp.zeros_like(acc_ref)

    t = t_ref[...]
    lane = lax.broadcasted_iota(jnp.int32, (_BR, 128), 1)
    wc = jnp.full((_BR, 128), _WC, dtype=jnp.float32)
    wf = jnp.full((_BR, 128), _WF, dtype=jnp.float32)
    accs = [jnp.zeros((_BR, 128), jnp.float32) for _ in range(_NACC)]
    for c in range(_NSL):
        col = lane + (c * 128)
        v = x_ref[:, c * 128:(c + 1) * 128]
        if (c + 1) * 128 > _C:
            v = jnp.where(col < _C, v, 0.0)
        w = jnp.where(col == t, wc, wf)
        accs[c % _NACC] = accs[c % _NACC] + v * w
    total = accs[0]
    for k in range(1, _NACC):
        total = total + accs[k]
    acc_ref[...] += total

    @pl.when(i < (_N - _NTC) // _BR)
    def _sc_rows_tail():
        colt = lane + _CTAIL
        vt = jnp.where(colt < _C, xt_ref[...], 0.0)
        wt = jnp.where(colt == tt_ref[...], wc, wf)
        acc_ref[...] += vt * wt

    @pl.when(i == _NRG - 1)
    def _final():
        o_ref[...] = jnp.sum(acc_ref[...]).reshape(1, 1)


def kernel(x, target):
    B, M, C = x.shape
    n = B * M
    x2 = x.reshape(n, C)
    t32 = target.reshape(n, 1).astype(jnp.int32)
    t128 = jnp.broadcast_to(t32, (n, 128))
    sc_out = _sc_part(x2[_NTC:], t128[_NTC:])
    ntg = _NTC // _BR
    nsg = (n - _NTC) // _BR - 1
    tile_tail = _CTAIL // 128
    tc_out = pl.pallas_call(
        _tc_body,
        grid=(_NRG,),
        in_specs=[
            pl.BlockSpec((_BR, 128), lambda i: (i, 0)),
            pl.BlockSpec((_BR, _BC), lambda i: (i, 0)),
            pl.BlockSpec((_BR, 128),
                         lambda i: (jnp.minimum(i, nsg) + ntg, 0)),
            pl.BlockSpec((_BR, 128),
                         lambda i: (jnp.minimum(i, nsg) + ntg, tile_tail)),
        ],
        out_specs=pl.BlockSpec((1, 1), lambda i: (0, 0)),
        out_shape=jax.ShapeDtypeStruct((1, 1), jnp.float32),
        scratch_shapes=[pltpu.VMEM((_BR, 128), jnp.float32)],
    )(t128, x2, t128, x2)
    return jnp.float32(_CONST) + tc_out[0, 0] + jnp.sum(sc_out)


# speedup vs baseline: 1.1707x; 1.1707x over previous
import functools
import math

import jax
import jax.numpy as jnp
from jax import lax
from jax.experimental import pallas as pl
from jax.experimental.pallas import tpu as pltpu
from jax.experimental.pallas import tpu_sc as plsc

_C = 100000
_N = 1024
_SMOOTHING = 0.1
_CONF = 1.0 - _SMOOTHING
_FILL = _SMOOTHING / (_C - 1)
_CONST = (_C - 1) * _FILL * math.log(_FILL) + _CONF * math.log(_CONF)
_WF = -_FILL / _N
_WC = -_CONF / _N

_NTC = 800
_NSC = _N - _NTC

_BR = 8
_NSL = (_C + 127) // 128
_BC = _NSL * 128
_NRG = _NTC // _BR
_NACC = 8

_NW = 32
_SRPW = _NSC // _NW
_GRPW = _N // _NW
_W0 = 49920
_W1 = _C - _W0
_NCH = _SRPW * 2
_UNROLL = 8


def _accum(buf, nwords, acc0, acc1):
    def body(k, carry):
        a0, a1 = carry
        base = k * (16 * _UNROLL)
        for u in range(_UNROLL):
            v = buf[pl.ds(base + u * 16, 16)]
            if u % 2 == 0:
                a0 = a0 + v
            else:
                a1 = a1 + v
        return (a0, a1)
    group = 16 * _UNROLL
    acc0, acc1 = lax.fori_loop(0, nwords // group, body, (acc0, acc1))
    for k in range((nwords % group) // 16):
        off = (nwords // group) * group + k * 16
        acc0 = acc0 + buf[pl.ds(off, 16)]
    return acc0, acc1


@functools.partial(
    pl.kernel,
    out_type=jax.ShapeDtypeStruct((_NW, 16), jnp.float32),
    scratch_types=[
        pltpu.VMEM((_W1,), jnp.float32),
        pltpu.VMEM((_W1,), jnp.float32),
        pltpu.VMEM((_GRPW, 16), jnp.int32),
        pltpu.VMEM((_GRPW, 16), jnp.float32),
        pltpu.VMEM((16,), jnp.float32),
        pltpu.SemaphoreType.DMA,
        pltpu.SemaphoreType.DMA,
        pltpu.SemaphoreType.DMA,
    ],
    mesh=plsc.VectorSubcoreMesh(core_axis_name="c", subcore_axis_name="s"),
)
def _sc_part(x_hbm, t16_hbm, out_hbm, buf0, buf1, idx_v, vals_v, accv,
             sem0, sem1, semg):
    wid = lax.axis_index("s") * 2 + lax.axis_index("c")
    r0 = _NTC + wid * _SRPW
    g0 = wid * _GRPW
    bufs = (buf0, buf1)
    sems = (sem0, sem1)

    pltpu.sync_copy(t16_hbm.at[pl.ds(g0, _GRPW)], idx_v)

    gcopies = []
    for r in range(_GRPW):
        t_s = idx_v[r][0]
        base = (t_s // 16) * 16
        gcopies.append(pltpu.async_copy(
            x_hbm.at[g0 + r, pl.ds(base, 16)],
            vals_v.at[r],
            semg))

    def chunk_src(c):
        row = r0 + c // 2
        if c % 2 == 0:
            return x_hbm.at[row, pl.ds(0, _W0)]
        return x_hbm.at[row, pl.ds(_W0, _W1)]

    def chunk_dst(c):
        w = _W0 if c % 2 == 0 else _W1
        return bufs[c % 2].at[pl.ds(0, w)]

    copies = {0: pltpu.async_copy(chunk_src(0), chunk_dst(0), sems[0])}
    acc0 = jnp.zeros((16,), jnp.float32)
    acc1 = jnp.zeros((16,), jnp.float32)
    for c in range(_NCH):
        if c + 1 < _NCH:
            copies[c + 1] = pltpu.async_copy(
                chunk_src(c + 1), chunk_dst(c + 1), sems[(c + 1) % 2])
        copies[c].wait()
        w = _W0 if c % 2 == 0 else _W1
        acc0, acc1 = _accum(bufs[c % 2], w, acc0, acc1)

    for cp in gcopies:
        cp.wait()
    lane16 = lax.iota(jnp.int32, 16)
    gacc = jnp.zeros((16,), jnp.float32)
    for r in range(_GRPW):
        loff = idx_v[r][0] % 16
        gacc = gacc + jnp.where(lane16 == loff, vals_v[r], 0.0)

    accv[...] = (jnp.float32(_WF) * (acc0 + acc1)
                 + jnp.float32(_WC - _WF) * gacc)
    pltpu.sync_copy(accv, out_hbm.at[wid])


def _tc_body(x_ref, o_ref, acc_ref):
    i = pl.program_id(0)

    @pl.when(i == 0)
    def _init():
        acc_ref[...] = jnp.zeros_like(acc_ref)

    lane = lax.broadcasted_iota(jnp.int32, (_BR, 128), 1)
    accs = [jnp.zeros((_BR, 128), jnp.float32) for _ in range(_NACC)]
    for c in range(_NSL):
        v = x_ref[:, c * 128:(c + 1) * 128]
        if (c + 1) * 128 > _C:
            v = jnp.where(lane + c * 128 < _C, v, 0.0)
        accs[c % _NACC] = accs[c % _NACC] + v
    total = accs[0]
    for k in range(1, _NACC):
        total = total + accs[k]
    acc_ref[...] += total

    @pl.when(i == _NRG - 1)
    def _final():
        o_ref[...] = jnp.sum(acc_ref[...]).reshape(1, 1)


def kernel(x, target):
    B, M, C = x.shape
    n = B * M
    x2 = x.reshape(n, C)
    t16 = jnp.broadcast_to(target.reshape(n, 1).astype(jnp.int32), (n, 16))
    sc_out = _sc_part(x2, t16)
    tc_out = pl.pallas_call(
        _tc_body,
        grid=(_NRG,),
        in_specs=[pl.BlockSpec((_BR, _BC), lambda i: (i, 0))],
        out_specs=pl.BlockSpec((1, 1), lambda i: (0, 0)),
        out_shape=jax.ShapeDtypeStruct((1, 1), jnp.float32),
        scratch_shapes=[pltpu.VMEM((_BR, 128), jnp.float32)],
    )(x2)
    return jnp.float32(_CONST) + jnp.float32(_WF) * tc_out[0, 0] + jnp.sum(sc_out)
